# baseline (speedup 1.0000x reference)
import jax
import jax.numpy as jnp
from jax import lax
from jax.experimental import pallas as pl

BT = 512
NROWS = 4096
CBN = 8192
CBD = 32
BETA = 0.25
GRID = NROWS // BT
HB = BT // 2
LOSS_SCALE = (1.0 + BETA) / (3.0 * NROWS * CBD)


def _mm(a, b):
    return lax.dot_general(a, b, (((1,), (0,)), ((), ())),
                           preferred_element_type=jnp.float32)


def _fused_kernel(x_ref, ew0, eb0, ew1, eb1, ew2, eb2, ew3, eb3,
                  dw0, db0, dw1, db1, dw2, db2, dw3, db3,
                  ch0, cl0, ch1, cl1, ch2, cl2, cm0, cm1, cm2,
                  out_ref, loss_ref, idx_ref):
    i = pl.program_id(0)
    cbs = ((ch0, cl0, cm0), (ch1, cl1, cm1), (ch2, cl2, cm2))
    cbsqs = [0.25 * jnp.sum(cm[...] * cm[...], axis=0, keepdims=True)
             for _, _, cm in cbs]
    cols = lax.broadcasted_iota(jnp.int32, (HB, CBN), 1)

    T = 2
    h = [x_ref[t * HB:(t + 1) * HB, :] for t in range(T)]
    for w, b, act in ((ew0, eb0, 1), (ew1, eb1, 1), (ew2, eb2, 1),
                      (ew3, eb3, 0)):
        for t in range(T):
            h[t] = _mm(h[t], w[...]) + b[...]
            if act:
                h[t] = jnp.maximum(h[t], 0.0)

    res = h
    xqsum = [jnp.zeros_like(res[t]) for t in range(T)]
    idx_parts = [[] for _ in range(T)]
    ssq = jnp.zeros((1, 1), jnp.float32)
    d = [None] * T; m = [None] * T; c = [None] * T
    idxv = [None] * T; onehot = [None] * T; xq = [None] * T
    for (chi, clo, cm2t), cbsq in zip(cbs, cbsqs):
        for t in range(T):
            rsq = jnp.sum(res[t] * res[t], axis=1, keepdims=True)
            d[t] = (rsq + cbsq) + _mm(res[t], cm2t[...])
        for t in range(T):
            m[t] = jnp.min(d[t], axis=1, keepdims=True)
        for t in range(T):
            c[t] = jnp.where(d[t] == m[t], cols, CBN)
        for t in range(T):
            idxv[t] = jnp.min(c[t], axis=1, keepdims=True)
        for t in range(T):
            onehot[t] = (c[t] == idxv[t]).astype(jnp.bfloat16)
        for t in range(T):
            xq[t] = _mm(onehot[t], chi[...]) + _mm(onehot[t], clo[...])
        for t in range(T):
            err = xq[t] - res[t]
            ssq = ssq + jnp.sum(err * err, keepdims=True)
            x_res = res[t] + (xq[t] - res[t])
            res[t] = res[t] - x_res
            xqsum[t] = xqsum[t] + x_res
            idx_parts[t].append(idxv[t])

    for t in range(T):
        idx_ref[t * HB:(t + 1) * HB, :] = jnp.concatenate(idx_parts[t],
                                                          axis=1)

    h = xqsum
    for w, b, act in ((dw0, db0, 1), (dw1, db1, 1), (dw2, db2, 1),
                      (dw3, db3, 0)):
        for t in range(T):
            h[t] = _mm(h[t], w[...]) + b[...]
            if act:
                h[t] = jnp.maximum(h[t], 0.0)
    for t in range(T):
        out_ref[t * HB:(t + 1) * HB, :] = h[t]

    prev = jnp.where(i == 0, jnp.zeros((1, 1), jnp.float32), loss_ref[...])
    tot = prev + ssq
    loss_ref[...] = jnp.where(i == GRID - 1, tot * LOSS_SCALE, tot)


def _full(shape):
    nd = len(shape)
    return pl.BlockSpec(shape, lambda i, _n=nd: (0,) * _n)


def kernel(x, enc_W0, enc_b0, enc_W1, enc_b1, enc_W2, enc_b2, enc_W3,
           enc_b3, dec_W0, dec_b0, dec_W1, dec_b1, dec_W2, dec_b2, dec_W3,
           dec_b3, cb0, cb1, cb2):
    ews = [enc_W0.T, enc_W1.T, enc_W2.T, enc_W3.T]
    ebs = [enc_b0.reshape(1, -1), enc_b1.reshape(1, -1),
           enc_b2.reshape(1, -1), enc_b3.reshape(1, -1)]
    dws = [dec_W0.T, dec_W1.T, dec_W2.T, dec_W3.T]
    dbs = [dec_b0.reshape(1, -1), dec_b1.reshape(1, -1),
           dec_b2.reshape(1, -1), dec_b3.reshape(1, -1)]
    cbm2ts = [(-2.0 * cb0).T, (-2.0 * cb1).T, (-2.0 * cb2).T]
    cb_his = [cb.astype(jnp.bfloat16) for cb in (cb0, cb1, cb2)]
    cb_los = [(cb - hi.astype(jnp.float32)).astype(jnp.bfloat16)
              for cb, hi in zip((cb0, cb1, cb2), cb_his)]

    operands = []
    in_specs = [pl.BlockSpec((BT, 768), lambda i: (i, 0))]
    operands.append(x)
    for w, b in zip(ews, ebs):
        operands += [w, b]
        in_specs += [_full(w.shape), _full(b.shape)]
    for w, b in zip(dws, dbs):
        operands += [w, b]
        in_specs += [_full(w.shape), _full(b.shape)]
    for hi, lo in zip(cb_his, cb_los):
        operands += [hi, lo]
        in_specs += [_full(hi.shape), _full(lo.shape)]
    for cm in cbm2ts:
        operands.append(cm)
        in_specs.append(_full(cm.shape))

    out, loss, idx = pl.pallas_call(
        _fused_kernel,
        grid=(GRID,),
        in_specs=in_specs,
        out_specs=[
            pl.BlockSpec((BT, 768), lambda i: (i, 0)),
            pl.BlockSpec((1, 1), lambda i: (0, 0)),
            pl.BlockSpec((BT, 3), lambda i: (i, 0)),
        ],
        out_shape=[
            jax.ShapeDtypeStruct((NROWS, 768), jnp.float32),
            jax.ShapeDtypeStruct((1, 1), jnp.float32),
            jax.ShapeDtypeStruct((NROWS, 3), jnp.int32),
        ],
    )(*operands)
    return (out, loss.reshape(()), idx)


# speedup vs baseline: 1.0537x; 1.0537x over previous
import jax
import jax.numpy as jnp
from jax import lax
from jax.experimental import pallas as pl

BT = 256
NROWS = 4096
CBN = 8192
CBD = 32
BETA = 0.25
GRID = NROWS // BT
HB = BT // 2
LOSS_SCALE = (1.0 + BETA) / (3.0 * NROWS * CBD)


def _mm(a, b):
    return lax.dot_general(a, b, (((1,), (0,)), ((), ())),
                           preferred_element_type=jnp.float32)


def _fused_kernel(x_ref, ew0, eb0, ew1, eb1, ew2, eb2, ew3, eb3,
                  dw0, db0, dw1, db1, dw2, db2, dw3, db3,
                  ch0, cl0, ch1, cl1, ch2, cl2, cm0, cm1, cm2, iota_row,
                  out_ref, loss_ref, idx_ref):
    i = pl.program_id(0)
    cbs = ((ch0, cl0, cm0), (ch1, cl1, cm1), (ch2, cl2, cm2))
    cbsqs = [0.25 * jnp.sum(cm[...] * cm[...], axis=0, keepdims=True)
             for _, _, cm in cbs]
    cols = iota_row[...]

    T = 2
    h = [x_ref[t * HB:(t + 1) * HB, :] for t in range(T)]
    for w, b, act in ((ew0, eb0, 1), (ew1, eb1, 1), (ew2, eb2, 1),
                      (ew3, eb3, 0)):
        for t in range(T):
            h[t] = _mm(h[t], w[...]) + b[...]
            if act:
                h[t] = jnp.maximum(h[t], 0.0)

    res = h
    xqsum = [jnp.zeros_like(res[t]) for t in range(T)]
    idx_parts = [[] for _ in range(T)]
    ssq = jnp.zeros((1, 1), jnp.float32)
    d = [None] * T; m = [None] * T; c = [None] * T
    idxv = [None] * T; onehot = [None] * T; xq = [None] * T
    for (chi, clo, cm2t), cbsq in zip(cbs, cbsqs):
        for t in range(T):
            rsq = jnp.sum(res[t] * res[t], axis=1, keepdims=True)
            d[t] = (rsq + cbsq) + _mm(res[t], cm2t[...])
        for t in range(T):
            m[t] = jnp.min(d[t], axis=1, keepdims=True)
        for t in range(T):
            c[t] = jnp.where(d[t] == m[t], cols, jnp.float32(CBN))
        for t in range(T):
            idxv[t] = jnp.min(c[t], axis=1, keepdims=True)
        for t in range(T):
            onehot[t] = (c[t] == idxv[t]).astype(jnp.bfloat16)
        for t in range(T):
            xq[t] = _mm(onehot[t], chi[...]) + _mm(onehot[t], clo[...])
        for t in range(T):
            err = xq[t] - res[t]
            ssq = ssq + jnp.sum(err * err, keepdims=True)
            x_res = res[t] + (xq[t] - res[t])
            res[t] = res[t] - x_res
            xqsum[t] = xqsum[t] + x_res
            idx_parts[t].append(idxv[t])

    for t in range(T):
        idx_ref[t * HB:(t + 1) * HB, :] = jnp.concatenate(
            idx_parts[t], axis=1).astype(jnp.int32)

    h = xqsum
    for w, b, act in ((dw0, db0, 1), (dw1, db1, 1), (dw2, db2, 1),
                      (dw3, db3, 0)):
        for t in range(T):
            h[t] = _mm(h[t], w[...]) + b[...]
            if act:
                h[t] = jnp.maximum(h[t], 0.0)
    for t in range(T):
        out_ref[t * HB:(t + 1) * HB, :] = h[t]

    prev = jnp.where(i == 0, jnp.zeros((1, 1), jnp.float32), loss_ref[...])
    tot = prev + ssq
    loss_ref[...] = jnp.where(i == GRID - 1, tot * LOSS_SCALE, tot)


def _full(shape):
    nd = len(shape)
    return pl.BlockSpec(shape, lambda i, _n=nd: (0,) * _n)


def kernel(x, enc_W0, enc_b0, enc_W1, enc_b1, enc_W2, enc_b2, enc_W3,
           enc_b3, dec_W0, dec_b0, dec_W1, dec_b1, dec_W2, dec_b2, dec_W3,
           dec_b3, cb0, cb1, cb2):
    ews = [enc_W0.T, enc_W1.T, enc_W2.T, enc_W3.T]
    ebs = [enc_b0.reshape(1, -1), enc_b1.reshape(1, -1),
           enc_b2.reshape(1, -1), enc_b3.reshape(1, -1)]
    dws = [dec_W0.T, dec_W1.T, dec_W2.T, dec_W3.T]
    dbs = [dec_b0.reshape(1, -1), dec_b1.reshape(1, -1),
           dec_b2.reshape(1, -1), dec_b3.reshape(1, -1)]
    cbm2ts = [(-2.0 * cb0).T, (-2.0 * cb1).T, (-2.0 * cb2).T]
    cb_his = [cb.astype(jnp.bfloat16) for cb in (cb0, cb1, cb2)]
    cb_los = [(cb - hi.astype(jnp.float32)).astype(jnp.bfloat16)
              for cb, hi in zip((cb0, cb1, cb2), cb_his)]

    operands = []
    in_specs = [pl.BlockSpec((BT, 768), lambda i: (i, 0))]
    operands.append(x)
    for w, b in zip(ews, ebs):
        operands += [w, b]
        in_specs += [_full(w.shape), _full(b.shape)]
    for w, b in zip(dws, dbs):
        operands += [w, b]
        in_specs += [_full(w.shape), _full(b.shape)]
    for hi, lo in zip(cb_his, cb_los):
        operands += [hi, lo]
        in_specs += [_full(hi.shape), _full(lo.shape)]
    for cm in cbm2ts:
        operands.append(cm)
        in_specs.append(_full(cm.shape))
    iota_row = jnp.arange(CBN, dtype=jnp.float32).reshape(1, CBN)
    operands.append(iota_row)
    in_specs.append(_full((1, CBN)))

    out, loss, idx = pl.pallas_call(
        _fused_kernel,
        grid=(GRID,),
        in_specs=in_specs,
        out_specs=[
            pl.BlockSpec((BT, 768), lambda i: (i, 0)),
            pl.BlockSpec((1, 1), lambda i: (0, 0)),
            pl.BlockSpec((BT, 3), lambda i: (i, 0)),
        ],
        out_shape=[
            jax.ShapeDtypeStruct((NROWS, 768), jnp.float32),
            jax.ShapeDtypeStruct((1, 1), jnp.float32),
            jax.ShapeDtypeStruct((NROWS, 3), jnp.int32),
        ],
    )(*operands)
    return (out, loss.reshape(()), idx)


# speedup vs baseline: 1.2807x; 1.2154x over previous
import jax
import jax.numpy as jnp
from jax import lax
from jax.experimental import pallas as pl

BT = 256
NROWS = 4096
CBN = 8192
CBD = 32
BETA = 0.25
GRID = NROWS // BT
HB = BT // 2
LOSS_SCALE = (1.0 + BETA) / (3.0 * NROWS * CBD)


def _mm(a, b):
    return lax.dot_general(a, b, (((1,), (0,)), ((), ())),
                           preferred_element_type=jnp.float32)


def _fused_kernel(x_ref, ew0, eb0, ew1, eb1, ew2, eb2, ew3, eb3,
                  dw0, db0, dw1, db1, dw2, db2, dw3, db3,
                  ch0, ch1, ch2, cm0, cm1, cm2, iota_row,
                  out_ref, loss_ref, idx_ref):
    i = pl.program_id(0)
    cbs = ((ch0, cm0), (ch1, cm1), (ch2, cm2))
    cbsqs = [0.25 * jnp.sum(cm[...] * cm[...], axis=0, keepdims=True)
             for _, cm in cbs]
    cols = iota_row[...]

    T = 2
    h = [x_ref[t * HB:(t + 1) * HB, :] for t in range(T)]
    for w, b, act in ((ew0, eb0, 1), (ew1, eb1, 1), (ew2, eb2, 1),
                      (ew3, eb3, 0)):
        for t in range(T):
            h[t] = _mm(h[t], w[...]) + b[...]
            if act:
                h[t] = jnp.maximum(h[t], 0.0)

    res = h
    xqsum = [jnp.zeros_like(res[t]) for t in range(T)]
    idx_parts = [[] for _ in range(T)]
    ssq = jnp.zeros((1, 1), jnp.float32)
    d = [None] * T; m = [None] * T; c = [None] * T
    idxv = [None] * T; onehot = [None] * T; xq = [None] * T
    for (chilo, cm2t), cbsq in zip(cbs, cbsqs):
        for t in range(T):
            rsq = jnp.sum(res[t] * res[t], axis=1, keepdims=True)
            d[t] = (rsq + cbsq) + _mm(res[t], cm2t[...])
        for t in range(T):
            m[t] = jnp.min(d[t], axis=1, keepdims=True)
        for t in range(T):
            c[t] = jnp.where(d[t] == m[t], cols, jnp.float32(CBN))
        for t in range(T):
            idxv[t] = jnp.min(c[t], axis=1, keepdims=True)
        for t in range(T):
            onehot[t] = (c[t] == idxv[t]).astype(jnp.bfloat16)
        for t in range(T):
            g = _mm(onehot[t], chilo[...])
            xq[t] = g[:, :CBD] + g[:, CBD:]
        for t in range(T):
            err = xq[t] - res[t]
            ssq = ssq + jnp.sum(err * err, keepdims=True)
            x_res = res[t] + (xq[t] - res[t])
            res[t] = res[t] - x_res
            xqsum[t] = xqsum[t] + x_res
            idx_parts[t].append(idxv[t])

    for t in range(T):
        idx_ref[t * HB:(t + 1) * HB, :] = jnp.concatenate(
            idx_parts[t], axis=1).astype(jnp.int32)

    h = xqsum
    for w, b, act in ((dw0, db0, 1), (dw1, db1, 1), (dw2, db2, 1),
                      (dw3, db3, 0)):
        for t in range(T):
            h[t] = _mm(h[t], w[...]) + b[...]
            if act:
                h[t] = jnp.maximum(h[t], 0.0)
    for t in range(T):
        out_ref[t * HB:(t + 1) * HB, :] = h[t]

    prev = jnp.where(i == 0, jnp.zeros((1, 1), jnp.float32), loss_ref[...])
    tot = prev + ssq
    loss_ref[...] = jnp.where(i == GRID - 1, tot * LOSS_SCALE, tot)


def _full(shape):
    nd = len(shape)
    return pl.BlockSpec(shape, lambda i, _n=nd: (0,) * _n)


def kernel(x, enc_W0, enc_b0, enc_W1, enc_b1, enc_W2, enc_b2, enc_W3,
           enc_b3, dec_W0, dec_b0, dec_W1, dec_b1, dec_W2, dec_b2, dec_W3,
           dec_b3, cb0, cb1, cb2):
    ews = [enc_W0.T, enc_W1.T, enc_W2.T, enc_W3.T]
    ebs = [enc_b0.reshape(1, -1), enc_b1.reshape(1, -1),
           enc_b2.reshape(1, -1), enc_b3.reshape(1, -1)]
    dws = [dec_W0.T, dec_W1.T, dec_W2.T, dec_W3.T]
    dbs = [dec_b0.reshape(1, -1), dec_b1.reshape(1, -1),
           dec_b2.reshape(1, -1), dec_b3.reshape(1, -1)]
    cbm2ts = [(-2.0 * cb0).T, (-2.0 * cb1).T, (-2.0 * cb2).T]
    cb_his = [cb.astype(jnp.bfloat16) for cb in (cb0, cb1, cb2)]
    cb_hilos = [jnp.concatenate(
        [hi, (cb - hi.astype(jnp.float32)).astype(jnp.bfloat16)], axis=1)
        for cb, hi in zip((cb0, cb1, cb2), cb_his)]

    operands = []
    in_specs = [pl.BlockSpec((BT, 768), lambda i: (i, 0))]
    operands.append(x)
    for w, b in zip(ews, ebs):
        operands += [w, b]
        in_specs += [_full(w.shape), _full(b.shape)]
    for w, b in zip(dws, dbs):
        operands += [w, b]
        in_specs += [_full(w.shape), _full(b.shape)]
    for hilo in cb_hilos:
        operands.append(hilo)
        in_specs.append(_full(hilo.shape))
    for cm in cbm2ts:
        operands.append(cm)
        in_specs.append(_full(cm.shape))
    iota_row = jnp.arange(CBN, dtype=jnp.float32).reshape(1, CBN)
    operands.append(iota_row)
    in_specs.append(_full((1, CBN)))

    out, loss, idx = pl.pallas_call(
        _fused_kernel,
        grid=(GRID,),
        in_specs=in_specs,
        out_specs=[
            pl.BlockSpec((BT, 768), lambda i: (i, 0)),
            pl.BlockSpec((1, 1), lambda i: (0, 0)),
            pl.BlockSpec((BT, 3), lambda i: (i, 0)),
        ],
        out_shape=[
            jax.ShapeDtypeStruct((NROWS, 768), jnp.float32),
            jax.ShapeDtypeStruct((1, 1), jnp.float32),
            jax.ShapeDtypeStruct((NROWS, 3), jnp.int32),
        ],
    )(*operands)
    return (out, loss.reshape(()), idx)


# speedup vs baseline: 1.4449x; 1.1283x over previous
import jax
import jax.numpy as jnp
from jax import lax
from jax.experimental import pallas as pl

BT = 256
NROWS = 4096
CBN = 8192
CBD = 32
BETA = 0.25
GRID = NROWS // BT
HB = BT // 2
LOSS_SCALE = (1.0 + BETA) / (3.0 * NROWS * CBD)


def _mm(a, b):
    return lax.dot_general(a, b, (((1,), (0,)), ((), ())),
                           preferred_element_type=jnp.float32)


def _fused_kernel(x_ref, ew0, eb0, ew1, eb1, ew2, eb2, ew3, eb3,
                  dw0, db0, dw1, db1, dw2, db2, dw3, db3,
                  ch0, ch1, ch2, cm0, cm1, cm2, iota_row,
                  out_ref, loss_ref, idx_ref):
    i = pl.program_id(0)
    cbs = ((ch0, cm0), (ch1, cm1), (ch2, cm2))
    cbsqs = [0.25 * jnp.sum(cm[...] * cm[...], axis=0, keepdims=True)
             for _, cm in cbs]
    cols = iota_row[...]

    T = 2
    h = [x_ref[t * HB:(t + 1) * HB, :] for t in range(T)]
    for w, b, act in ((ew0, eb0, 1), (ew1, eb1, 1), (ew2, eb2, 1),
                      (ew3, eb3, 0)):
        for t in range(T):
            h[t] = _mm(h[t], w[...]) + b[...]
            if act:
                h[t] = jnp.maximum(h[t], 0.0)

    res = h
    xqsum = [jnp.zeros_like(res[t]) for t in range(T)]
    idx_parts = [[] for _ in range(T)]
    ssq = jnp.zeros((1, 1), jnp.float32)
    d = [None] * T; m = [None] * T; c = [None] * T
    idxv = [None] * T; onehot = [None] * T; xq = [None] * T
    for (chilo, cm2t), cbsq in zip(cbs, cbsqs):
        for t in range(T):
            rsq = jnp.sum(res[t] * res[t], axis=1, keepdims=True)
            d[t] = (rsq + cbsq) + _mm(res[t], cm2t[...])
        for t in range(T):
            idxv[t] = jnp.argmin(d[t], axis=1)[:, None]
        for t in range(T):
            onehot[t] = (cols == idxv[t].astype(jnp.float32)
                         ).astype(jnp.bfloat16)
        for t in range(T):
            g = _mm(onehot[t], chilo[...])
            xq[t] = g[:, :CBD] + g[:, CBD:]
        for t in range(T):
            err = xq[t] - res[t]
            ssq = ssq + jnp.sum(err * err, keepdims=True)
            x_res = res[t] + (xq[t] - res[t])
            res[t] = res[t] - x_res
            xqsum[t] = xqsum[t] + x_res
            idx_parts[t].append(idxv[t])

    for t in range(T):
        idx_ref[t * HB:(t + 1) * HB, :] = jnp.concatenate(
            idx_parts[t], axis=1)

    h = xqsum
    for w, b, act in ((dw0, db0, 1), (dw1, db1, 1), (dw2, db2, 1),
                      (dw3, db3, 0)):
        for t in range(T):
            h[t] = _mm(h[t], w[...]) + b[...]
            if act:
                h[t] = jnp.maximum(h[t], 0.0)
    for t in range(T):
        out_ref[t * HB:(t + 1) * HB, :] = h[t]

    prev = jnp.where(i == 0, jnp.zeros((1, 1), jnp.float32), loss_ref[...])
    tot = prev + ssq
    loss_ref[...] = jnp.where(i == GRID - 1, tot * LOSS_SCALE, tot)


def _full(shape):
    nd = len(shape)
    return pl.BlockSpec(shape, lambda i, _n=nd: (0,) * _n)


def kernel(x, enc_W0, enc_b0, enc_W1, enc_b1, enc_W2, enc_b2, enc_W3,
           enc_b3, dec_W0, dec_b0, dec_W1, dec_b1, dec_W2, dec_b2, dec_W3,
           dec_b3, cb0, cb1, cb2):
    ews = [enc_W0.T, enc_W1.T, enc_W2.T, enc_W3.T]
    ebs = [enc_b0.reshape(1, -1), enc_b1.reshape(1, -1),
           enc_b2.reshape(1, -1), enc_b3.reshape(1, -1)]
    dws = [dec_W0.T, dec_W1.T, dec_W2.T, dec_W3.T]
    dbs = [dec_b0.reshape(1, -1), dec_b1.reshape(1, -1),
           dec_b2.reshape(1, -1), dec_b3.reshape(1, -1)]
    cbm2ts = [(-2.0 * cb0).T, (-2.0 * cb1).T, (-2.0 * cb2).T]
    cb_his = [cb.astype(jnp.bfloat16) for cb in (cb0, cb1, cb2)]
    cb_hilos = [jnp.concatenate(
        [hi, (cb - hi.astype(jnp.float32)).astype(jnp.bfloat16)], axis=1)
        for cb, hi in zip((cb0, cb1, cb2), cb_his)]

    operands = []
    in_specs = [pl.BlockSpec((BT, 768), lambda i: (i, 0))]
    operands.append(x)
    for w, b in zip(ews, ebs):
        operands += [w, b]
        in_specs += [_full(w.shape), _full(b.shape)]
    for w, b in zip(dws, dbs):
        operands += [w, b]
        in_specs += [_full(w.shape), _full(b.shape)]
    for hilo in cb_hilos:
        operands.append(hilo)
        in_specs.append(_full(hilo.shape))
    for cm in cbm2ts:
        operands.append(cm)
        in_specs.append(_full(cm.shape))
    iota_row = jnp.arange(CBN, dtype=jnp.float32).reshape(1, CBN)
    operands.append(iota_row)
    in_specs.append(_full((1, CBN)))

    out, loss, idx = pl.pallas_call(
        _fused_kernel,
        grid=(GRID,),
        in_specs=in_specs,
        out_specs=[
            pl.BlockSpec((BT, 768), lambda i: (i, 0)),
            pl.BlockSpec((1, 1), lambda i: (0, 0)),
            pl.BlockSpec((BT, 3), lambda i: (i, 0)),
        ],
        out_shape=[
            jax.ShapeDtypeStruct((NROWS, 768), jnp.float32),
            jax.ShapeDtypeStruct((1, 1), jnp.float32),
            jax.ShapeDtypeStruct((NROWS, 3), jnp.int32),
        ],
    )(*operands)
    return (out, loss.reshape(()), idx)
